# baseline (speedup 1.0000x reference)
import jax
import jax.numpy as jnp
from jax.experimental import pallas as pl
from jax.experimental.pallas import tpu as pltpu
from jax.experimental.pallas import tpu_sc as plsc

B = 4096
L = 50
EMB = 128

BLOCK_B = 8
CHUNKS = 4
CHUNK_B = B // CHUNKS


def _sc_gather_chunk(table, idx):
    mesh = plsc.VectorSubcoreMesh(core_axis_name="core",
                                  subcore_axis_name="subcore")

    @pl.kernel(out_type=jax.ShapeDtypeStruct((CHUNK_B, L, EMB), table.dtype),
               mesh=mesh,
               scratch_types=[pltpu.SemaphoreType.DMA])
    def gather_kernel(table_hbm, i_hbm, o_hbm, sem):
        def body(i_vmem, o_vmem):
            copies = [
                pltpu.make_async_copy(table_hbm.at[i_vmem.at[b]],
                                      o_vmem.at[b], sem)
                for b in range(BLOCK_B)
            ]
            for c in copies:
                c.start()
            for c in copies:
                c.wait()

        pltpu.emit_pipeline(
            body,
            grid=(CHUNK_B // BLOCK_B,),
            in_specs=[pl.BlockSpec((BLOCK_B, L), index_map=lambda i: (i, 0))],
            out_specs=[pl.BlockSpec((BLOCK_B, L, EMB),
                                    index_map=lambda i: (i, 0, 0))],
            core_axis_name=("core", "subcore"),
            dimension_semantics=(pltpu.PARALLEL,),
        )(i_hbm, o_hbm)

    return gather_kernel(table, idx)


def kernel(x, table):
    xi = x.astype(jnp.int32)
    outs = [
        _sc_gather_chunk(table, jax.lax.slice(xi, (k * CHUNK_B, 0),
                                              ((k + 1) * CHUNK_B, L)))
        for k in range(CHUNKS)
    ]
    return jnp.concatenate(outs, axis=0)
